# Initial kernel scaffold; baseline (speedup 1.0000x reference)
import jax
import jax.numpy as jnp
from jax.experimental import pallas as pl


def kernel(seq_samples, reward_mask):
    raise NotImplementedError("write your pallas kernel here")


# speedup vs baseline: 578.3989x; 578.3989x over previous
import jax
import jax.numpy as jnp
from jax import lax
from jax.experimental import pallas as pl
from jax.experimental.pallas import tpu as pltpu
from jax.experimental.pallas import tpu_sc as plsc

NC, NS, L = 2, 16, 16
NW = NC * NS
ROWS = 8 * 16384
ROW_LEN = 200
ROWS_PER_W = ROWS // NW
CHUNK_ROWS = 64
N_CHUNKS = ROWS_PER_W // CHUNK_ROWS
GROUPS = CHUNK_ROWS // L
MASK_PAD = 24


def _sc_body(mask_hbm, seq_hbm, out_hbm, mask_v, buf_v, out_v, sem):
    wid = lax.axis_index("s") * NC + lax.axis_index("c")
    base_row = wid * ROWS_PER_W
    pltpu.sync_copy(mask_hbm, mask_v)
    lane_off = lax.iota(jnp.int32, L) * ROW_LEN

    def chunk_body(c, tok):
        chunk_base = pl.multiple_of((base_row + c * CHUNK_ROWS) * ROW_LEN, 8)
        pltpu.async_copy(
            seq_hbm.at[pl.ds(chunk_base, CHUNK_ROWS * ROW_LEN)], buf_v, sem
        ).wait()
        for g in range(GROUPS):
            goff = lane_off + g * (L * ROW_LEN)

            def j_body(j, acc):
                idx = plsc.load_gather(buf_v, [goff + j])
                vals = plsc.load_gather(mask_v, [idx])
                return acc + vals

            acc = lax.fori_loop(
                0, ROW_LEN, j_body, jnp.zeros((L,), jnp.float32), unroll=8
            )
            dst = pl.multiple_of(c * CHUNK_ROWS + g * L, 8)
            out_v[pl.ds(dst, L)] = acc * (1.0 / ROW_LEN)
        return tok

    lax.fori_loop(0, N_CHUNKS, chunk_body, 0)
    pltpu.sync_copy(out_v, out_hbm.at[pl.ds(base_row * 1, ROWS_PER_W)])


def kernel(seq_samples, reward_mask):
    seq_flat = seq_samples.reshape(-1)
    mask_p = jnp.pad(reward_mask, (0, MASK_PAD - reward_mask.shape[0]))
    mesh = plsc.VectorSubcoreMesh(
        core_axis_name="c", subcore_axis_name="s", num_cores=NC, num_subcores=NS
    )
    out = pl.kernel(
        _sc_body,
        out_type=jax.ShapeDtypeStruct((ROWS,), jnp.float32),
        mesh=mesh,
        compiler_params=pltpu.CompilerParams(needs_layout_passes=False),
        scratch_types=[
            pltpu.VMEM((MASK_PAD,), jnp.float32),
            pltpu.VMEM((CHUNK_ROWS * ROW_LEN,), jnp.int32),
            pltpu.VMEM((ROWS_PER_W,), jnp.float32),
            pltpu.SemaphoreType.DMA,
        ],
    )(mask_p, seq_flat)
    return out.reshape(seq_samples.shape[:2])


# speedup vs baseline: 657.9815x; 1.1376x over previous
import jax
import jax.numpy as jnp
from jax import lax
from jax.experimental import pallas as pl
from jax.experimental.pallas import tpu as pltpu
from jax.experimental.pallas import tpu_sc as plsc

NC, NS, L = 2, 16, 16
NW = NC * NS
ROWS = 8 * 16384
ROW_LEN = 200
ROWS_PER_W = ROWS // NW
CHUNK_ROWS = 64
CHUNK_ELEMS = CHUNK_ROWS * ROW_LEN
N_CHUNKS = ROWS_PER_W // CHUNK_ROWS
GROUPS = CHUNK_ROWS // L
N_OUTER = N_CHUNKS // 2
MASK_PAD = 24
INV_LEN = 1.0 / ROW_LEN


def _sc_body(mask_hbm, seq_hbm, out_hbm, mask_v, buf0, buf1, out_v, sem0, sem1):
    wid = lax.axis_index("s") * NC + lax.axis_index("c")
    base_row = wid * ROWS_PER_W
    base_elem = base_row * ROW_LEN
    pltpu.sync_copy(mask_hbm, mask_v)
    lane_off = lax.iota(jnp.int32, L) * ROW_LEN
    goffs = [lane_off + g * (L * ROW_LEN) for g in range(GROUPS)]
    zeros = jnp.zeros((L,), jnp.float32)

    def start(c, buf, sem):
        src = pl.multiple_of(base_elem + c * CHUNK_ELEMS, 8)
        pltpu.async_copy(seq_hbm.at[pl.ds(src, CHUNK_ELEMS)], buf, sem)

    def compute(buf, c):
        def j_body(j, accs):
            out = []
            for g in range(GROUPS):
                idx = plsc.load_gather(buf, [goffs[g] + j])
                out.append(accs[g] + plsc.load_gather(mask_v, [idx]))
            return tuple(out)

        accs = lax.fori_loop(
            0, ROW_LEN, j_body, (zeros,) * GROUPS, unroll=8
        )
        for g in range(GROUPS):
            dst = pl.multiple_of(c * CHUNK_ROWS + g * L, 8)
            out_v[pl.ds(dst, L)] = accs[g] * INV_LEN

    start(0, buf0, sem0)
    start(1, buf1, sem1)

    def outer(cc, tok):
        c0 = cc * 2
        pltpu.make_async_copy(
            seq_hbm.at[pl.ds(0, CHUNK_ELEMS)], buf0, sem0
        ).wait()
        compute(buf0, c0)

        @pl.when(cc < N_OUTER - 1)
        def _():
            start(c0 + 2, buf0, sem0)

        pltpu.make_async_copy(
            seq_hbm.at[pl.ds(0, CHUNK_ELEMS)], buf1, sem1
        ).wait()
        compute(buf1, c0 + 1)

        @pl.when(cc < N_OUTER - 1)
        def _():
            start(c0 + 3, buf1, sem1)

        return tok

    lax.fori_loop(0, N_OUTER, outer, 0)
    pltpu.sync_copy(out_v, out_hbm.at[pl.ds(base_row, ROWS_PER_W)])


def kernel(seq_samples, reward_mask):
    seq_flat = seq_samples.reshape(-1)
    mask_p = jnp.pad(reward_mask, (0, MASK_PAD - reward_mask.shape[0]))
    mesh = plsc.VectorSubcoreMesh(
        core_axis_name="c", subcore_axis_name="s", num_cores=NC, num_subcores=NS
    )
    out = pl.kernel(
        _sc_body,
        out_type=jax.ShapeDtypeStruct((ROWS,), jnp.float32),
        mesh=mesh,
        compiler_params=pltpu.CompilerParams(needs_layout_passes=False),
        scratch_types=[
            pltpu.VMEM((MASK_PAD,), jnp.float32),
            pltpu.VMEM((CHUNK_ELEMS,), jnp.int32),
            pltpu.VMEM((CHUNK_ELEMS,), jnp.int32),
            pltpu.VMEM((ROWS_PER_W,), jnp.float32),
            pltpu.SemaphoreType.DMA,
            pltpu.SemaphoreType.DMA,
        ],
    )(mask_p, seq_flat)
    return out.reshape(seq_samples.shape[:2])


# speedup vs baseline: 722.9950x; 1.0988x over previous
import jax
import jax.numpy as jnp
from jax import lax
from jax.experimental import pallas as pl
from jax.experimental.pallas import tpu as pltpu
from jax.experimental.pallas import tpu_sc as plsc

NC, NS, L = 2, 16, 16
NW = NC * NS
ROWS = 8 * 16384
ROW_LEN = 200
ROWS_PER_W = ROWS // NW
CHUNK_ROWS = 64
CHUNK_ELEMS = CHUNK_ROWS * ROW_LEN
N_CHUNKS = ROWS_PER_W // CHUNK_ROWS
GROUPS = CHUNK_ROWS // L
N_OUTER = N_CHUNKS // 2
MASK_PAD = 24
INV_LEN = 1.0 / ROW_LEN


def _sc_body(mask_hbm, seq_hbm, out_hbm, mask_v, buf0, buf1, out_v, sem0, sem1):
    wid = lax.axis_index("s") * NC + lax.axis_index("c")
    base_row = wid * ROWS_PER_W
    base_elem = base_row * ROW_LEN
    pltpu.sync_copy(mask_hbm, mask_v)
    lane = lax.iota(jnp.int32, L)
    starts = [lane * (ROW_LEN + 1) + g * (L * ROW_LEN) for g in range(GROUPS)]
    row_ends = [lane * ROW_LEN + ROW_LEN + g * (L * ROW_LEN) for g in range(GROUPS)]
    zeros = jnp.zeros((L,), jnp.float32)

    def start(c, buf, sem):
        src = pl.multiple_of(base_elem + c * CHUNK_ELEMS, 8)
        pltpu.async_copy(seq_hbm.at[pl.ds(src, CHUNK_ELEMS)], buf, sem)

    def compute(buf, c):
        def j_body(j, accs):
            out = []
            for g in range(GROUPS):
                a0 = starts[g] + j
                addr = jnp.where(a0 >= row_ends[g], a0 - ROW_LEN, a0)
                idx = plsc.load_gather(buf, [addr])
                vals = plsc.load_gather(mask_v, [idx * L + lane])
                out.append(accs[g] + vals)
            return tuple(out)

        accs = lax.fori_loop(
            0, ROW_LEN, j_body, (zeros,) * GROUPS, unroll=8
        )
        for g in range(GROUPS):
            dst = pl.multiple_of(c * CHUNK_ROWS + g * L, 8)
            out_v[pl.ds(dst, L)] = accs[g] * INV_LEN

    start(0, buf0, sem0)
    start(1, buf1, sem1)

    def outer(cc, tok):
        c0 = cc * 2
        pltpu.make_async_copy(
            seq_hbm.at[pl.ds(0, CHUNK_ELEMS)], buf0, sem0
        ).wait()
        compute(buf0, c0)

        @pl.when(cc < N_OUTER - 1)
        def _():
            start(c0 + 2, buf0, sem0)

        pltpu.make_async_copy(
            seq_hbm.at[pl.ds(0, CHUNK_ELEMS)], buf1, sem1
        ).wait()
        compute(buf1, c0 + 1)

        @pl.when(cc < N_OUTER - 1)
        def _():
            start(c0 + 3, buf1, sem1)

        return tok

    lax.fori_loop(0, N_OUTER, outer, 0)
    pltpu.sync_copy(out_v, out_hbm.at[pl.ds(base_row, ROWS_PER_W)])


def kernel(seq_samples, reward_mask):
    seq_flat = seq_samples.reshape(-1)
    mask_p = jnp.repeat(
        jnp.pad(reward_mask, (0, MASK_PAD - reward_mask.shape[0])), L
    )
    mesh = plsc.VectorSubcoreMesh(
        core_axis_name="c", subcore_axis_name="s", num_cores=NC, num_subcores=NS
    )
    out = pl.kernel(
        _sc_body,
        out_type=jax.ShapeDtypeStruct((ROWS,), jnp.float32),
        mesh=mesh,
        compiler_params=pltpu.CompilerParams(needs_layout_passes=False),
        scratch_types=[
            pltpu.VMEM((MASK_PAD * L,), jnp.float32),
            pltpu.VMEM((CHUNK_ELEMS,), jnp.int32),
            pltpu.VMEM((CHUNK_ELEMS,), jnp.int32),
            pltpu.VMEM((ROWS_PER_W,), jnp.float32),
            pltpu.SemaphoreType.DMA,
            pltpu.SemaphoreType.DMA,
        ],
    )(mask_p, seq_flat)
    return out.reshape(seq_samples.shape[:2])


# speedup vs baseline: 1888.4966x; 2.6120x over previous
import jax
import jax.numpy as jnp
from jax import lax
from jax.experimental import pallas as pl
from jax.experimental.pallas import tpu as pltpu
from jax.experimental.pallas import tpu_sc as plsc

NC, NS, L = 2, 16, 16
NW = NC * NS
B, S, J = 8, 16384, 200
JT, JR = 25, 8
ST, SR = 128, 128
ROWS = B * S
ROWS_PER_W = ROWS // NW
ST_PER_W = ROWS_PER_W // SR
CHUNK_WORDS = ST_PER_W * JR * SR
PAIRS = JR // 2
M2_PAD = 704
INV_LEN = 1.0 / J


def _sc_body(mask2_hbm, seq_hbm, out_hbm, mask2_v, buf0, buf1, acc_v, sem0, sem1):
    wid = lax.axis_index("s") * NC + lax.axis_index("c")
    b = wid // 4
    st0 = (wid % 4) * ST_PER_W
    pltpu.sync_copy(mask2_hbm, mask2_v)
    lane = lax.iota(jnp.int32, L)
    zeros = jnp.zeros((L,), jnp.float32)

    def zero_body(i, tok):
        acc_v[pl.ds(pl.multiple_of(i * L, 8), L)] = zeros
        return tok

    lax.fori_loop(0, ROWS_PER_W // L, zero_body, 0)

    def start(jt, buf, sem):
        src = pl.multiple_of(((b * JT + jt) * ST + st0) * (JR * SR), 8)
        pltpu.async_copy(seq_hbm.at[pl.ds(src, CHUNK_WORDS)], buf, sem)

    def wait(buf, sem):
        pltpu.make_async_copy(
            seq_hbm.at[pl.ds(0, CHUNK_WORDS)], buf, sem
        ).wait()

    def compute(buf):
        def st_body(st, tok):
            sbase = st * (JR * SR)
            abase = st * SR
            for q in range(JR):
                aoff = pl.multiple_of(abase + q * L, 8)
                acc = acc_v[pl.ds(aoff, L)]
                for p in range(PAIRS):
                    offa = pl.multiple_of(sbase + (2 * p) * SR + q * L, 8)
                    offb = pl.multiple_of(sbase + (2 * p + 1) * SR + q * L, 8)
                    a = buf[pl.ds(offa, L)]
                    bb = buf[pl.ds(offb, L)]
                    addr = ((a << 5) + bb) * L + lane
                    acc = acc + plsc.load_gather(mask2_v, [addr])
                acc_v[pl.ds(aoff, L)] = acc
            return tok

        lax.fori_loop(0, ST_PER_W, st_body, 0)

    start(0, buf0, sem0)
    start(1, buf1, sem1)

    def outer(cc, tok):
        jt = cc * 2
        wait(buf0, sem0)
        compute(buf0)

        @pl.when(cc < JT // 2)
        def _():
            start(jt + 2, buf0, sem0)

        wait(buf1, sem1)
        compute(buf1)

        @pl.when(cc < JT // 2 - 1)
        def _():
            start(jt + 3, buf1, sem1)

        return tok

    lax.fori_loop(0, JT // 2, outer, 0)
    wait(buf0, sem0)
    compute(buf0)

    def scale_body(i, tok):
        off = pl.multiple_of(i * L, 8)
        acc_v[pl.ds(off, L)] = acc_v[pl.ds(off, L)] * INV_LEN
        return tok

    lax.fori_loop(0, ROWS_PER_W // L, scale_body, 0)
    pltpu.sync_copy(acc_v, out_hbm.at[pl.ds(wid * ROWS_PER_W, ROWS_PER_W)])


def kernel(seq_samples, reward_mask):
    seq_view = (
        seq_samples.transpose(0, 2, 1)
        .reshape(B, JT, JR, ST, SR)
        .transpose(0, 1, 3, 2, 4)
        .reshape(-1)
    )
    m2 = reward_mask[:, None] + reward_mask[None, :]
    m2 = jnp.pad(m2, ((0, 1), (0, 32 - m2.shape[1])))
    mask2 = jnp.repeat(m2.reshape(-1), L)
    mesh = plsc.VectorSubcoreMesh(
        core_axis_name="c", subcore_axis_name="s", num_cores=NC, num_subcores=NS
    )
    out = pl.kernel(
        _sc_body,
        out_type=jax.ShapeDtypeStruct((ROWS,), jnp.float32),
        mesh=mesh,
        compiler_params=pltpu.CompilerParams(needs_layout_passes=False),
        scratch_types=[
            pltpu.VMEM((M2_PAD * L,), jnp.float32),
            pltpu.VMEM((CHUNK_WORDS,), jnp.int32),
            pltpu.VMEM((CHUNK_WORDS,), jnp.int32),
            pltpu.VMEM((ROWS_PER_W,), jnp.float32),
            pltpu.SemaphoreType.DMA,
            pltpu.SemaphoreType.DMA,
        ],
    )(mask2, seq_view)
    return out.reshape(B, S)
